# baseline (speedup 1.0000x reference)
import functools

import jax
import jax.numpy as jnp
from jax import lax
from jax.experimental import pallas as pl
from jax.experimental.pallas import tpu as pltpu
from jax.experimental.pallas import tpu_sc as plsc

TRIGGER = 0.5
N_NODES = 100000
D_FEAT = 128
W_UPD = 32
B = 1024

NUM_CORES = 2
NUM_SUBCORES = 16
LANES = 16
NUM_WORKERS = NUM_CORES * NUM_SUBCORES
REGION_ROWS = 3200
N_PAD = REGION_ROWS * NUM_WORKERS
NUM_CP_VREGS = B // LANES
SLOT_WORDS = NUM_CP_VREGS * LANES + LANES

_mesh = plsc.VectorSubcoreMesh(core_axis_name="c", subcore_axis_name="s")


@functools.partial(
    pl.kernel,
    out_type=jax.ShapeDtypeStruct((N_PAD,), jnp.float32),
    mesh=_mesh,
    compiler_params=pltpu.CompilerParams(needs_layout_passes=False),
    scratch_types=[
        pltpu.VMEM((B,), jnp.int32),
        pltpu.VMEM((REGION_ROWS + LANES,), jnp.float32),
        pltpu.VMEM((SLOT_WORDS,), jnp.int32),
        pltpu.VMEM((SLOT_WORDS,), jnp.int32),
    ],
)
def _sc_bias(cp_hbm, bias_hbm, cp_v, acc, slot_c, slot_m):
    wid = lax.axis_index("s") * NUM_CORES + lax.axis_index("c")
    base = wid * REGION_ROWS

    pltpu.sync_copy(cp_hbm, cp_v)

    zerosf = jnp.zeros((LANES,), jnp.float32)

    def zero_body(i, carry):
        acc[pl.ds(i * LANES, LANES)] = zerosf
        return carry

    lax.fori_loop(0, (REGION_ROWS + LANES) // LANES, zero_body, 0)

    lo_v = jnp.full((LANES,), base, jnp.int32)
    hi_v = jnp.full((LANES,), base + REGION_ROWS, jnp.int32)
    ones = jnp.ones((LANES,), jnp.int32)
    zeros = jnp.zeros((LANES,), jnp.int32)

    def scan_body(v, cnt):
        c_vec = cp_v[pl.ds(v * LANES, LANES)]
        m_ge = jnp.where(c_vec >= lo_v, ones, zeros)
        m_lt = jnp.where(c_vec < hi_v, ones, zeros)
        m_i32 = m_ge * m_lt
        n_match = jnp.sum(m_i32)
        slot_c[pl.ds(cnt * LANES, LANES)] = c_vec
        slot_m[pl.ds(cnt * LANES, LANES)] = m_i32
        return cnt + jnp.where(n_match > 0, 1, 0)

    cnt = lax.fori_loop(0, NUM_CP_VREGS, scan_body, 0)

    half0 = jnp.where(lax.iota(jnp.int32, LANES) == 0,
                      jnp.full((LANES,), TRIGGER, jnp.float32), zerosf)

    def slot_body(s, carry):
        def lane_body(l, carry2):
            off = s * LANES + l
            mval = slot_m[pl.ds(off, LANES)][0]

            @pl.when(mval > 0)
            def _():
                r = slot_c[pl.ds(off, LANES)][0] - base
                acc[pl.ds(r, LANES)] = acc[pl.ds(r, LANES)] + half0
            return carry2

        lax.fori_loop(0, LANES, lane_body, 0)
        return carry

    lax.fori_loop(0, cnt, slot_body, 0)

    pltpu.sync_copy(acc.at[pl.ds(0, REGION_ROWS)],
                    bias_hbm.at[pl.ds(base, REGION_ROWS)])


TC_BLOCK_ROWS = 1024
TC_GRID = -(-N_NODES // TC_BLOCK_ROWS)


def _tc_body(x_ref, bias_ref, out_ref):
    xb = x_ref[...]
    bb = bias_ref[...]
    lane = lax.broadcasted_iota(jnp.int32, xb.shape, 1)
    upd = jnp.minimum(xb + bb, 1.0)
    out_ref[...] = jnp.where((lane < W_UPD) & (bb > 0.0), upd, xb)


_tc_apply = pl.pallas_call(
    _tc_body,
    grid=(TC_GRID,),
    in_specs=[
        pl.BlockSpec((TC_BLOCK_ROWS, D_FEAT), lambda i: (i, 0)),
        pl.BlockSpec((TC_BLOCK_ROWS, 1), lambda i: (i, 0)),
    ],
    out_specs=pl.BlockSpec((TC_BLOCK_ROWS, D_FEAT), lambda i: (i, 0)),
    out_shape=jax.ShapeDtypeStruct((N_NODES, D_FEAT), jnp.float32),
)


def kernel(x, center, ptr):
    cp = (center + ptr[:-1]).astype(jnp.int32)
    bias = _sc_bias(cp)[:N_NODES].reshape(N_NODES, 1)
    return _tc_apply(x, bias)


# speedup vs baseline: 2.8530x; 2.8530x over previous
import functools

import jax
import jax.numpy as jnp
from jax import lax
from jax.experimental import pallas as pl
from jax.experimental.pallas import tpu as pltpu
from jax.experimental.pallas import tpu_sc as plsc

TRIGGER = 0.5
N_NODES = 100000
D_FEAT = 128
B = 1024

NUM_CORES = 2
NUM_SUBCORES = 16
LANES = 16
NUM_WORKERS = NUM_CORES * NUM_SUBCORES
CHUNK_ROWS = 200
CHUNK_ELEMS = CHUNK_ROWS * D_FEAT
NUM_CHUNKS = N_NODES // CHUNK_ROWS
CHUNKS_PER_WORKER = -(-NUM_CHUNKS // NUM_WORKERS)
NBUF = 3
NUM_STEPS = -(-CHUNKS_PER_WORKER // NBUF) * NBUF
NUM_CP_VREGS = B // LANES
SLOT_WORDS = NUM_CP_VREGS * LANES + LANES

_mesh = plsc.VectorSubcoreMesh(core_axis_name="c", subcore_axis_name="s")


@functools.partial(
    pl.kernel,
    out_type=jax.ShapeDtypeStruct((N_NODES * D_FEAT,), jnp.float32),
    mesh=_mesh,
    compiler_params=pltpu.CompilerParams(needs_layout_passes=False),
    scratch_types=[
        pltpu.VMEM((B,), jnp.int32),
        pltpu.VMEM((CHUNK_ELEMS,), jnp.float32),
        pltpu.VMEM((CHUNK_ELEMS,), jnp.float32),
        pltpu.VMEM((CHUNK_ELEMS,), jnp.float32),
        pltpu.VMEM((SLOT_WORDS,), jnp.int32),
        pltpu.VMEM((SLOT_WORDS,), jnp.int32),
        pltpu.SemaphoreType.DMA,
        pltpu.SemaphoreType.DMA,
        pltpu.SemaphoreType.DMA,
        pltpu.SemaphoreType.DMA,
        pltpu.SemaphoreType.DMA,
        pltpu.SemaphoreType.DMA,
    ],
)
def _sc_copy_update(x_hbm, cp_hbm, out_hbm, cp_v, buf0, buf1, buf2,
                    slot_c, slot_m, isem0, isem1, isem2, osem0, osem1, osem2):
    wid = lax.axis_index("s") * NUM_CORES + lax.axis_index("c")

    pltpu.sync_copy(cp_hbm, cp_v)

    slots = ((buf0, isem0, osem0), (buf1, isem1, osem1), (buf2, isem2, osem2))

    def ci_of(k):
        return wid + k * NUM_WORKERS

    def valid(k):
        ci = ci_of(k)
        return (k >= 0) & (ci < NUM_CHUNKS)

    def start_in(k, buf, isem):
        @pl.when(valid(k))
        def _():
            e0 = ci_of(k) * CHUNK_ELEMS
            pltpu.make_async_copy(
                x_hbm.at[pl.ds(e0, CHUNK_ELEMS)], buf, isem).start()

    def wait_in(k, buf, isem):
        @pl.when(valid(k))
        def _():
            pltpu.make_async_copy(
                x_hbm.at[pl.ds(0, CHUNK_ELEMS)], buf, isem).wait()

    def start_out(k, buf, osem):
        @pl.when(valid(k))
        def _():
            e0 = ci_of(k) * CHUNK_ELEMS
            pltpu.make_async_copy(
                buf, out_hbm.at[pl.ds(e0, CHUNK_ELEMS)], osem).start()

    def wait_out(k, buf, osem):
        @pl.when(valid(k))
        def _():
            pltpu.make_async_copy(
                buf, out_hbm.at[pl.ds(0, CHUNK_ELEMS)], osem).wait()

    def scan_chunk(k):
        row0 = ci_of(k) * CHUNK_ROWS
        lo_v = jnp.full((LANES,), row0, jnp.int32)
        hi_v = jnp.full((LANES,), row0 + CHUNK_ROWS, jnp.int32)
        ones = jnp.ones((LANES,), jnp.int32)
        zeros = jnp.zeros((LANES,), jnp.int32)

        def scan_body(v, cnt):
            c_vec = cp_v[pl.ds(v * LANES, LANES)]
            m_ge = jnp.where(c_vec >= lo_v, ones, zeros)
            m_lt = jnp.where(c_vec < hi_v, ones, zeros)
            m_i32 = m_ge * m_lt
            n_match = jnp.sum(m_i32)
            slot_c[pl.ds(cnt * LANES, LANES)] = c_vec
            slot_m[pl.ds(cnt * LANES, LANES)] = m_i32
            return cnt + jnp.where(n_match > 0, 1, 0)

        return lax.fori_loop(0, NUM_CP_VREGS, scan_body, 0)

    def apply_chunk(k, cnt, buf):
        @pl.when(valid(k))
        def _():
            row0 = ci_of(k) * CHUNK_ROWS

            def slot_body(s, carry):
                def lane_body(l, carry2):
                    off = s * LANES + l
                    mval = slot_m[pl.ds(off, LANES)][0]

                    @pl.when(mval > 0)
                    def _():
                        r = slot_c[pl.ds(off, LANES)][0] - row0
                        e = r * D_FEAT
                        s0 = buf[pl.ds(e, LANES)]
                        buf[pl.ds(e, LANES)] = jnp.minimum(s0 + TRIGGER, 1.0)
                        s1 = buf[pl.ds(e + LANES, LANES)]
                        buf[pl.ds(e + LANES, LANES)] = jnp.minimum(s1 + TRIGGER, 1.0)
                    return carry2

                lax.fori_loop(0, LANES, lane_body, 0)
                return carry

            lax.fori_loop(0, cnt, slot_body, 0)

    def step(k, t):
        buf, isem, osem = slots[t]
        nbuf, nisem, nosem = slots[(t + 1) % NBUF]
        wait_out(k + 1 - NBUF, nbuf, nosem)
        start_in(k + 1, nbuf, nisem)
        cnt = scan_chunk(k)
        wait_in(k, buf, isem)
        apply_chunk(k, cnt, buf)
        start_out(k, buf, osem)

    start_in(0, buf0, isem0)

    def body(j, carry):
        k0 = NBUF * j
        for t in range(NBUF):
            step(k0 + t, t)
        return carry

    lax.fori_loop(0, NUM_STEPS // NBUF, body, 0)


def kernel(x, center, ptr):
    cp = (center + ptr[:-1]).astype(jnp.int32)
    flat = _sc_copy_update(x.reshape(N_NODES * D_FEAT), cp)
    return flat.reshape(N_NODES, D_FEAT)
